# baseline (speedup 1.0000x reference)
import functools

import jax
import jax.numpy as jnp
from jax import lax
from jax.experimental import pallas as pl
from jax.experimental.pallas import tpu as pltpu
from jax.experimental.pallas import tpu_sc as plsc

LANES = 16
CS = 16


def _make_kernel(batch, seq, dim):
    info = plsc.get_sparse_core_info()
    nc, ns = info.num_cores, info.num_subcores
    nw = nc * ns
    seq_per_w = seq // nw
    nchunk = seq_per_w // CS
    npair = nchunk // 2
    vecs_per_row = dim // LANES
    nb2 = batch // 2

    mesh = plsc.VectorSubcoreMesh(core_axis_name="c", subcore_axis_name="s")

    buf_t = pltpu.VMEM((CS, dim), jnp.float32)

    @functools.partial(
        pl.kernel,
        mesh=mesh,
        out_type=jax.ShapeDtypeStruct((batch, seq, dim), jnp.float32),
        scratch_types=(
            [buf_t] * 2
            + [buf_t] * 2
            + [buf_t, buf_t]
            + [pltpu.SemaphoreType.DMA] * 2
            + [pltpu.SemaphoreType.DMA] * 2
            + [pltpu.SemaphoreType.DMA] * 4
        ),
    )
    def k(in_hbm, pos_hbm, out_hbm, *scratch):
        bufs = (scratch[0:2], scratch[2:4])
        pos_v = (scratch[4], scratch[5])
        sem_in = (scratch[6], scratch[7])
        sem_pos = (scratch[8], scratch[9])
        sem_out = (scratch[10:12], scratch[12:14])

        wid = lax.axis_index("s") * nc + lax.axis_index("c")
        seq0 = wid * seq_per_w

        def issue_ins(s0, st):
            for j in range(nb2):
                pltpu.async_copy(
                    in_hbm.at[st * nb2 + j, pl.ds(s0, CS)], bufs[st][j],
                    sem_in[st])

        def wait_ins(s0, st):
            for j in range(nb2):
                pltpu.make_async_copy(
                    in_hbm.at[st * nb2 + j, pl.ds(s0, CS)], bufs[st][j],
                    sem_in[st]).wait()

        def issue_pos(s0, pp):
            pltpu.async_copy(pos_hbm.at[pl.ds(s0, CS)], pos_v[pp], sem_pos[pp])

        def wait_pos(s0, pp):
            pltpu.make_async_copy(
                pos_hbm.at[pl.ds(s0, CS)], pos_v[pp], sem_pos[pp]).wait()

        def add_rows(st, j, pp):
            buf = bufs[st][j]
            pv = pos_v[pp]

            def row_body(r, _):
                for c in range(vecs_per_row):
                    sl = pl.ds(c * LANES, LANES)
                    plsc.addupdate(buf.at[r, sl], pv[r, sl])
                return _

            lax.fori_loop(0, CS, row_body, 0)

        def phase(s0, st, pp, reload_pred):
            so = st ^ 1
            reload_s0 = s0 if st == 0 else s0 + CS
            wait_ins(s0, st)
            for j in range(nb2):
                add_rows(st, j, pp)
                pltpu.async_copy(
                    bufs[st][j], out_hbm.at[st * nb2 + j, pl.ds(s0, CS)],
                    sem_out[st][j])

                @pl.when(reload_pred)
                def _r():
                    pltpu.make_async_copy(
                        bufs[so][j],
                        out_hbm.at[so * nb2 + j, pl.ds(s0, CS)],
                        sem_out[so][j]).wait()
                    pltpu.async_copy(
                        in_hbm.at[so * nb2 + j, pl.ds(reload_s0, CS)],
                        bufs[so][j], sem_in[so])

        def chunk(c_idx, s0, pp, first, last):
            if not last:
                @pl.when(c_idx < nchunk - 1)
                def _pp():
                    issue_pos(s0 + CS, pp ^ 1)

            @pl.when(c_idx > 0)
            def _wp():
                if not first:
                    wait_pos(s0, pp)

            phase(s0, 0, pp, c_idx > 0)
            phase(s0, 1, pp, c_idx < nchunk - 1)

        issue_pos(seq0, 0)
        issue_ins(seq0, 0)
        issue_ins(seq0, 1)
        wait_pos(seq0, 0)

        def pair_body(ci, _):
            c0 = 2 * ci
            s0 = seq0 + c0 * CS
            chunk(c0, s0, 0, first=(False), last=False)
            chunk(c0 + 1, s0 + CS, 1, first=False, last=False)
            return _

        lax.fori_loop(0, npair, pair_body, 0)

        tail = seq0 + (nchunk - 1) * CS
        for st in range(2):
            for j in range(nb2):
                pltpu.make_async_copy(
                    bufs[st][j],
                    out_hbm.at[st * nb2 + j, pl.ds(tail, CS)],
                    sem_out[st][j]).wait()

    return k


def kernel(inputs, position_embeddings):
    batch, seq, dim = inputs.shape
    k = _make_kernel(batch, seq, dim)
    return k(inputs, position_embeddings)
